# baseline (speedup 1.0000x reference)
import functools

import jax
import jax.numpy as jnp
from jax import lax
from jax.experimental import pallas as pl
from jax.experimental.pallas import tpu as pltpu
from jax.experimental.pallas import tpu_sc as plsc

VOCAB = 100000
HID = 128
NCLS = 128
B = 4096
L = 50

NC = 2
NS = 16
NW = NC * NS
BPW = B // NW
NCH = NCLS // 16

RPB = 1
NBLK = BPW // RPB
NBUF = 2
NITER = NBLK // NBUF

ROWS_BLK = 2000


def _transform_body(t_ref, w_ref, b_ref, z_ref):
    z = jnp.dot(t_ref[...], w_ref[...], preferred_element_type=jnp.float32)
    z_ref[...] = jnp.exp(jnp.maximum(z + b_ref[...], 0.0))


def _transform_table(table, W, b):
    return pl.pallas_call(
        _transform_body,
        grid=(VOCAB // ROWS_BLK,),
        in_specs=[
            pl.BlockSpec((ROWS_BLK, HID), lambda i: (i, 0)),
            pl.BlockSpec((HID, NCLS), lambda i: (0, 0)),
            pl.BlockSpec((1, NCLS), lambda i: (0, 0)),
        ],
        out_specs=pl.BlockSpec((ROWS_BLK, NCLS), lambda i: (i, 0)),
        out_shape=jax.ShapeDtypeStruct((VOCAB, NCLS), jnp.float32),
    )(table, W, b.reshape(1, NCLS))


def _normalize(gbuf, obuf, off):
    def sum_body(l, accs):
        return tuple(accs[c] + gbuf[l, pl.ds(c * 16, 16)]
                     for c in range(NCH))

    zeros = tuple(jnp.zeros((16,), jnp.float32) for _ in range(NCH))
    accs = lax.fori_loop(off, off + L, sum_body, zeros)
    invs = tuple(1.0 / a for a in accs)

    def norm_body(l, carry):
        for c in range(NCH):
            obuf[pl.ds(l * NCLS + c * 16, 16)] = (
                gbuf[l, pl.ds(c * 16, 16)] * invs[c])
        return carry

    lax.fori_loop(off, off + L, norm_body, 0)


def _gather_softmax_kernel(x_hbm, z_hbm, out_hbm, idx_v, *scratch):
    gbufs = scratch[:NBUF]
    obufs = scratch[NBUF:2 * NBUF]
    gsems = scratch[2 * NBUF:3 * NBUF]
    wsems = scratch[3 * NBUF:]

    wid = lax.axis_index("s") * NC + lax.axis_index("c")
    base = wid * NBLK
    pltpu.sync_copy(x_hbm.at[pl.ds(base, NBLK), :], idx_v)

    def gather(k, j):
        return pltpu.make_async_copy(z_hbm.at[idx_v.at[k]], gbufs[j],
                                     gsems[j])

    def write(k, j):
        return pltpu.make_async_copy(obufs[j], out_hbm.at[base + k],
                                     wsems[j])

    for j in range(NBUF):
        gather(j, j).start()

    def body(i2, carry):
        for j in range(NBUF):
            k = i2 * NBUF + j
            gather(k, j).wait()

            @pl.when(i2 > 0)
            def _():
                write(k - NBUF, j).wait()

            for r in range(RPB):
                _normalize(gbufs[j], obufs[j], r * L)
            write(k, j).start()

            @pl.when(i2 + 1 < NITER)
            def _():
                gather(k + NBUF, j).start()

        return carry

    lax.fori_loop(0, NITER, body, 0)

    for j in range(NBUF):
        write((NITER - 1) * NBUF + j, j).wait()


@functools.cache
def _gather_softmax():
    return pl.kernel(
        _gather_softmax_kernel,
        out_type=jax.ShapeDtypeStruct((B // RPB, RPB * L * NCLS),
                                      jnp.float32),
        mesh=plsc.VectorSubcoreMesh(core_axis_name="c", subcore_axis_name="s"),
        scratch_types=[
            pltpu.VMEM((NBLK, RPB * L), jnp.int32),
            *[pltpu.VMEM((RPB * L, NCLS), jnp.float32) for _ in range(NBUF)],
            *[pltpu.VMEM((RPB * L * NCLS,), jnp.float32)
              for _ in range(NBUF)],
            *[pltpu.SemaphoreType.DMA for _ in range(2 * NBUF)],
        ],
    )


def kernel(x, embed_table, W, b):
    z = _transform_table(embed_table, W, b)
    y = _gather_softmax()(x.reshape(B // RPB, RPB * L), z)
    return y.reshape(B, L, NCLS)


# speedup vs baseline: 1.3685x; 1.3685x over previous
import functools

import jax
import jax.numpy as jnp
from jax import lax
from jax.experimental import pallas as pl
from jax.experimental.pallas import tpu as pltpu
from jax.experimental.pallas import tpu_sc as plsc

VOCAB = 100000
HID = 128
NCLS = 128
B = 4096
L = 50

NC = 2
NS = 16
NW = NC * NS
BPW = B // NW
NCH = NCLS // 16

RPB = 1
NBLK = BPW // RPB
NBUF = 2
NITER = NBLK // NBUF

ROWS_BLK = 2000


def _transform_body(t_ref, w_ref, b_ref, z_ref):
    z = jnp.dot(t_ref[...], w_ref[...], preferred_element_type=jnp.float32)
    z_ref[...] = jnp.exp(jnp.maximum(z + b_ref[...], 0.0))


def _transform_table(table, W, b):
    return pl.pallas_call(
        _transform_body,
        grid=(VOCAB // ROWS_BLK,),
        in_specs=[
            pl.BlockSpec((ROWS_BLK, HID), lambda i: (i, 0)),
            pl.BlockSpec((HID, NCLS), lambda i: (0, 0)),
            pl.BlockSpec((1, NCLS), lambda i: (0, 0)),
        ],
        out_specs=pl.BlockSpec((ROWS_BLK, NCLS), lambda i: (i, 0)),
        out_shape=jax.ShapeDtypeStruct((VOCAB, NCLS), jnp.float32),
    )(table, W, b.reshape(1, NCLS))


def _normalize(gbuf, obuf, off):
    def sum_body(l, accs):
        return tuple(accs[c] + gbuf[l, pl.ds(c * 16, 16)]
                     for c in range(NCH))

    zeros = tuple(jnp.zeros((16,), jnp.float32) for _ in range(NCH))
    accs = lax.fori_loop(off, off + L, sum_body, zeros)
    invs = tuple(1.0 / a for a in accs)

    def norm_body(l, carry):
        for c in range(NCH):
            obuf[l, pl.ds(c * 16, 16)] = gbuf[l, pl.ds(c * 16, 16)] * invs[c]
        return carry

    lax.fori_loop(off, off + L, norm_body, 0)


def _gather_softmax_kernel(x_hbm, z_hbm, out_hbm, idx_v, *scratch):
    gbufs = scratch[:NBUF]
    obufs = scratch[NBUF:2 * NBUF]
    gsems = scratch[2 * NBUF:3 * NBUF]
    wsems = scratch[3 * NBUF:]

    wid = lax.axis_index("s") * NC + lax.axis_index("c")
    base = wid * NBLK
    pltpu.sync_copy(x_hbm.at[pl.ds(base, NBLK), :], idx_v)

    def gather(k, j):
        return pltpu.make_async_copy(z_hbm.at[idx_v.at[k]], gbufs[j],
                                     gsems[j])

    def write(k, j):
        return pltpu.make_async_copy(obufs[j], out_hbm.at[base + k],
                                     wsems[j])

    for j in range(NBUF):
        gather(j, j).start()

    def body(i2, carry):
        for j in range(NBUF):
            k = i2 * NBUF + j
            gather(k, j).wait()

            @pl.when(i2 > 0)
            def _():
                write(k - NBUF, j).wait()

            for r in range(RPB):
                _normalize(gbufs[j], obufs[j], r * L)
            write(k, j).start()

            @pl.when(i2 + 1 < NITER)
            def _():
                gather(k + NBUF, j).start()

        return carry

    lax.fori_loop(0, NITER, body, 0)

    for j in range(NBUF):
        write((NITER - 1) * NBUF + j, j).wait()


@functools.cache
def _gather_softmax():
    return pl.kernel(
        _gather_softmax_kernel,
        out_type=jax.ShapeDtypeStruct((B // RPB, RPB * L, NCLS),
                                      jnp.float32),
        mesh=plsc.VectorSubcoreMesh(core_axis_name="c", subcore_axis_name="s"),
        scratch_types=[
            pltpu.VMEM((NBLK, RPB * L), jnp.int32),
            *[pltpu.VMEM((RPB * L, NCLS), jnp.float32)
              for _ in range(2 * NBUF)],
            *[pltpu.SemaphoreType.DMA for _ in range(2 * NBUF)],
        ],
    )


def kernel(x, embed_table, W, b):
    z = _transform_table(embed_table, W, b)
    y = _gather_softmax()(x.reshape(B // RPB, RPB * L), z)
    return y.reshape(B, L, NCLS)


# speedup vs baseline: 1.5693x; 1.1467x over previous
import functools

import jax
import jax.numpy as jnp
from jax import lax
from jax.experimental import pallas as pl
from jax.experimental.pallas import tpu as pltpu
from jax.experimental.pallas import tpu_sc as plsc

VOCAB = 100000
HID = 128
NCLS = 128
B = 4096
L = 50

NC = 2
NS = 16
NW = NC * NS
BPW = B // NW
NCH = NCLS // 16

RPB = 1
NBLK = BPW // RPB
NBUF = 4
NITER = NBLK // NBUF

ROWS_BLK = 2000


def _transform_body(t_ref, w_ref, b_ref, z_ref):
    z = jnp.dot(t_ref[...], w_ref[...], preferred_element_type=jnp.float32)
    z_ref[...] = jnp.exp(jnp.maximum(z + b_ref[...], 0.0))


def _transform_table(table, W, b):
    return pl.pallas_call(
        _transform_body,
        grid=(VOCAB // ROWS_BLK,),
        in_specs=[
            pl.BlockSpec((ROWS_BLK, HID), lambda i: (i, 0)),
            pl.BlockSpec((HID, NCLS), lambda i: (0, 0)),
            pl.BlockSpec((1, NCLS), lambda i: (0, 0)),
        ],
        out_specs=pl.BlockSpec((ROWS_BLK, NCLS), lambda i: (i, 0)),
        out_shape=jax.ShapeDtypeStruct((VOCAB, NCLS), jnp.float32),
    )(table, W, b.reshape(1, NCLS))


def _normalize(gbuf, obuf, off):
    def sum_body(l, accs):
        return tuple(accs[c] + gbuf[l, pl.ds(c * 16, 16)]
                     for c in range(NCH))

    zeros = tuple(jnp.zeros((16,), jnp.float32) for _ in range(NCH))
    accs = lax.fori_loop(off, off + L, sum_body, zeros)
    invs = tuple(1.0 / a for a in accs)

    def norm_body(l, carry):
        for c in range(NCH):
            obuf[l, pl.ds(c * 16, 16)] = gbuf[l, pl.ds(c * 16, 16)] * invs[c]
        return carry

    lax.fori_loop(off, off + L, norm_body, 0)


def _gather_softmax_kernel(x_hbm, z_hbm, out_hbm, idx_v, *scratch):
    gbufs = scratch[:NBUF]
    obufs = scratch[NBUF:2 * NBUF]
    gsems = scratch[2 * NBUF:3 * NBUF]
    wsems = scratch[3 * NBUF:]

    wid = lax.axis_index("s") * NC + lax.axis_index("c")
    base = wid * NBLK
    pltpu.sync_copy(x_hbm.at[pl.ds(base, NBLK), :], idx_v)

    def gather(k, j):
        return pltpu.make_async_copy(z_hbm.at[idx_v.at[k]], gbufs[j],
                                     gsems[j])

    def write(k, j):
        return pltpu.make_async_copy(obufs[j], out_hbm.at[base + k],
                                     wsems[j])

    for j in range(NBUF):
        gather(j, j).start()

    def body(i2, carry):
        for j in range(NBUF):
            k = i2 * NBUF + j
            gather(k, j).wait()

            @pl.when(i2 > 0)
            def _():
                write(k - NBUF, j).wait()

            for r in range(RPB):
                _normalize(gbufs[j], obufs[j], r * L)
            write(k, j).start()

            @pl.when(i2 + 1 < NITER)
            def _():
                gather(k + NBUF, j).start()

        return carry

    lax.fori_loop(0, NITER, body, 0)

    for j in range(NBUF):
        write((NITER - 1) * NBUF + j, j).wait()


@functools.cache
def _gather_softmax():
    return pl.kernel(
        _gather_softmax_kernel,
        out_type=jax.ShapeDtypeStruct((B // RPB, RPB * L, NCLS),
                                      jnp.float32),
        mesh=plsc.VectorSubcoreMesh(core_axis_name="c", subcore_axis_name="s"),
        scratch_types=[
            pltpu.VMEM((NBLK, RPB * L), jnp.int32),
            *[pltpu.VMEM((RPB * L, NCLS), jnp.float32)
              for _ in range(2 * NBUF)],
            *[pltpu.SemaphoreType.DMA for _ in range(2 * NBUF)],
        ],
    )


def kernel(x, embed_table, W, b):
    z = _transform_table(embed_table, W, b)
    y = _gather_softmax()(x.reshape(B // RPB, RPB * L), z)
    return y.reshape(B, L, NCLS)


# speedup vs baseline: 1.5941x; 1.0158x over previous
import functools

import jax
import jax.numpy as jnp
from jax import lax
from jax.experimental import pallas as pl
from jax.experimental.pallas import tpu as pltpu
from jax.experimental.pallas import tpu_sc as plsc

VOCAB = 100000
HID = 128
NCLS = 128
B = 4096
L = 50

NC = 2
NS = 16
NW = NC * NS
BPW = B // NW
NCH = NCLS // 16

RPB = 1
NBLK = BPW // RPB
NBUF = 8
NITER = NBLK // NBUF

ROWS_BLK = 2000


def _transform_body(t_ref, w_ref, b_ref, z_ref):
    z = jnp.dot(t_ref[...], w_ref[...], preferred_element_type=jnp.float32)
    z_ref[...] = jnp.exp(jnp.maximum(z + b_ref[...], 0.0))


def _transform_table(table, W, b):
    return pl.pallas_call(
        _transform_body,
        grid=(VOCAB // ROWS_BLK,),
        in_specs=[
            pl.BlockSpec((ROWS_BLK, HID), lambda i: (i, 0)),
            pl.BlockSpec((HID, NCLS), lambda i: (0, 0)),
            pl.BlockSpec((1, NCLS), lambda i: (0, 0)),
        ],
        out_specs=pl.BlockSpec((ROWS_BLK, NCLS), lambda i: (i, 0)),
        out_shape=jax.ShapeDtypeStruct((VOCAB, NCLS), jnp.float32),
    )(table, W, b.reshape(1, NCLS))


def _normalize(gbuf, obuf, off):
    def sum_body(l, accs):
        return tuple(accs[c] + gbuf[l, pl.ds(c * 16, 16)]
                     for c in range(NCH))

    zeros = tuple(jnp.zeros((16,), jnp.float32) for _ in range(NCH))
    accs = lax.fori_loop(off, off + L, sum_body, zeros)
    invs = tuple(1.0 / a for a in accs)

    def norm_body(l, carry):
        for c in range(NCH):
            obuf[l, pl.ds(c * 16, 16)] = gbuf[l, pl.ds(c * 16, 16)] * invs[c]
        return carry

    lax.fori_loop(off, off + L, norm_body, 0)


def _gather_softmax_kernel(x_hbm, z_hbm, out_hbm, idx_v, *scratch):
    gbufs = scratch[:NBUF]
    obufs = scratch[NBUF:2 * NBUF]
    gsems = scratch[2 * NBUF:3 * NBUF]
    wsems = scratch[3 * NBUF:]

    wid = lax.axis_index("s") * NC + lax.axis_index("c")
    base = wid * NBLK
    pltpu.sync_copy(x_hbm.at[pl.ds(base, NBLK), :], idx_v)

    def gather(k, j):
        return pltpu.make_async_copy(z_hbm.at[idx_v.at[k]], gbufs[j],
                                     gsems[j])

    def write(k, j):
        return pltpu.make_async_copy(obufs[j], out_hbm.at[base + k],
                                     wsems[j])

    for j in range(NBUF):
        gather(j, j).start()

    def body(i2, carry):
        for j in range(NBUF):
            k = i2 * NBUF + j
            gather(k, j).wait()

            @pl.when(i2 > 0)
            def _():
                write(k - NBUF, j).wait()

            for r in range(RPB):
                _normalize(gbufs[j], obufs[j], r * L)
            write(k, j).start()

            @pl.when(i2 + 1 < NITER)
            def _():
                gather(k + NBUF, j).start()

        return carry

    lax.fori_loop(0, NITER, body, 0)

    for j in range(NBUF):
        write((NITER - 1) * NBUF + j, j).wait()


@functools.cache
def _gather_softmax():
    return pl.kernel(
        _gather_softmax_kernel,
        out_type=jax.ShapeDtypeStruct((B // RPB, RPB * L, NCLS),
                                      jnp.float32),
        mesh=plsc.VectorSubcoreMesh(core_axis_name="c", subcore_axis_name="s"),
        scratch_types=[
            pltpu.VMEM((NBLK, RPB * L), jnp.int32),
            *[pltpu.VMEM((RPB * L, NCLS), jnp.float32)
              for _ in range(2 * NBUF)],
            *[pltpu.SemaphoreType.DMA for _ in range(2 * NBUF)],
        ],
    )


def kernel(x, embed_table, W, b):
    z = _transform_table(embed_table, W, b)
    y = _gather_softmax()(x.reshape(B // RPB, RPB * L), z)
    return y.reshape(B, L, NCLS)
